# Initial kernel scaffold; baseline (speedup 1.0000x reference)
import jax
import jax.numpy as jnp
from jax.experimental import pallas as pl


def kernel(h, edge_index, e):
    raise NotImplementedError("write your pallas kernel here")


# speedup vs baseline: 2.7524x; 2.7524x over previous
import functools

import jax
import jax.numpy as jnp
from jax import lax
from jax.experimental import pallas as pl
from jax.experimental.pallas import tpu as pltpu
from jax.experimental.pallas import tpu_sc as plsc

N = 10000
E = 320000
D = 128

NC = 2
NS = 16
FS = NS
W = D // FS
EPC = E // NC
CHUNK = 8000
NCH = EPC // CHUNK
L = 16


def _sc_body(hg_hbm, src_hbm, dst_hbm, part_hbm, hs, src_v, dst_v, out_v):
    c = lax.axis_index("c")
    s = lax.axis_index("s")
    pltpu.sync_copy(hg_hbm.at[s], hs)

    def chunk_body(k, _):
        off = pl.multiple_of((c * NCH + k) * CHUNK, CHUNK)
        pltpu.sync_copy(src_hbm.at[pl.ds(off, CHUNK)], src_v)
        pltpu.sync_copy(dst_hbm.at[pl.ds(off, CHUNK)], dst_v)

        def grp(i, _):
            b = pl.multiple_of(i * L, L)
            si = src_v[pl.ds(b, L)] * W
            di = dst_v[pl.ds(b, L)] * W
            acc = jnp.zeros((L,), jnp.float32)
            for j in range(W):
                a = plsc.load_gather(hs, [si + j])
                bv = plsc.load_gather(hs, [di + j])
                acc = acc + a * bv
            out_v[pl.ds(b, L)] = acc
            return 0

        lax.fori_loop(0, CHUNK // L, grp, 0, unroll=2)
        pltpu.sync_copy(out_v, part_hbm.at[s, c * NCH + k])
        return 0

    lax.fori_loop(0, NCH, chunk_body, 0)


def _sc_partials(h_grouped, src, dst):
    mesh = plsc.VectorSubcoreMesh(core_axis_name="c", subcore_axis_name="s")
    k = pl.kernel(
        _sc_body,
        out_type=jax.ShapeDtypeStruct((FS, NC * NCH, CHUNK), jnp.float32),
        mesh=mesh,
        compiler_params=pltpu.CompilerParams(needs_layout_passes=False),
        scratch_types=[
            pltpu.VMEM((N * W,), jnp.float32),
            pltpu.VMEM((CHUNK,), jnp.int32),
            pltpu.VMEM((CHUNK,), jnp.int32),
            pltpu.VMEM((CHUNK,), jnp.float32),
        ],
    )
    return k(h_grouped, src, dst)


def _tc_reduce_body(p_ref, o_ref):
    o_ref[...] = jnp.sum(p_ref[...], axis=0, keepdims=True)


def _tc_reduce(partial2d):
    tcb = 6400
    return pl.pallas_call(
        _tc_reduce_body,
        grid=(E // tcb,),
        in_specs=[pl.BlockSpec((FS, tcb), lambda i: (0, i))],
        out_specs=pl.BlockSpec((1, tcb), lambda i: (0, i)),
        out_shape=jax.ShapeDtypeStruct((1, E), jnp.float32),
    )(partial2d)


@jax.jit
def kernel(h, edge_index, e):
    del e
    h_grouped = h.reshape(N, FS, W).transpose(1, 0, 2).reshape(FS, N * W)
    src = edge_index[0]
    dst = edge_index[1]
    partial = _sc_partials(h_grouped, src, dst)
    score = _tc_reduce(partial.reshape(FS, E))
    return score.reshape(E, 1)


# speedup vs baseline: 3.7797x; 1.3732x over previous
import jax
import jax.numpy as jnp
from jax import lax
from jax.experimental import pallas as pl
from jax.experimental.pallas import tpu as pltpu
from jax.experimental.pallas import tpu_sc as plsc

N = 10000
E = 320000
D = 128

FS = 8
W = 8
ES = 4
EPW = E // ES
CHUNK = 8000
NCH = EPW // CHUNK
L = 16


def _sc_body(hg_hbm, src_hbm, dst_hbm, part_hbm, hs, src_v, dst_v, out_v):
    c = lax.axis_index("c")
    s = lax.axis_index("s")
    fg = s % FS
    eg = c * 2 + s // FS
    pltpu.sync_copy(hg_hbm.at[fg], hs)

    def chunk_body(k, _):
        ch = eg * NCH + k
        off = pl.multiple_of(ch * CHUNK, CHUNK)
        pltpu.sync_copy(src_hbm.at[pl.ds(off, CHUNK)], src_v)
        pltpu.sync_copy(dst_hbm.at[pl.ds(off, CHUNK)], dst_v)

        def grp(i, _):
            b = pl.multiple_of(i * L, L)
            si = src_v[pl.ds(b, L)] * W
            di = dst_v[pl.ds(b, L)] * W
            acc = jnp.zeros((L,), jnp.float32)
            for j in range(W):
                wa = plsc.load_gather(hs, [si + j])
                wb = plsc.load_gather(hs, [di + j])
                a2 = plsc.bitcast(wa, jnp.bfloat16)
                b2 = plsc.bitcast(wb, jnp.bfloat16)
                p = a2 * b2
                pa, pb = plsc.unpack(
                    p,
                    format=plsc.PackFormat.INTERLEAVED,
                    preferred_element_type=jnp.float32,
                )
                acc = acc + pa + pb
            out_v[pl.ds(b, L)] = acc
            return 0

        lax.fori_loop(0, CHUNK // L, grp, 0, unroll=2)
        pltpu.sync_copy(out_v, part_hbm.at[fg, ch])
        return 0

    lax.fori_loop(0, NCH, chunk_body, 0)


def _sc_partials(h_grouped, src, dst):
    mesh = plsc.VectorSubcoreMesh(core_axis_name="c", subcore_axis_name="s")
    k = pl.kernel(
        _sc_body,
        out_type=jax.ShapeDtypeStruct((FS, ES * NCH, CHUNK), jnp.float32),
        mesh=mesh,
        compiler_params=pltpu.CompilerParams(needs_layout_passes=False),
        scratch_types=[
            pltpu.VMEM((N * W,), jnp.int32),
            pltpu.VMEM((CHUNK,), jnp.int32),
            pltpu.VMEM((CHUNK,), jnp.int32),
            pltpu.VMEM((CHUNK,), jnp.float32),
        ],
    )
    return k(h_grouped, src, dst)


def _tc_reduce_body(p_ref, o_ref):
    o_ref[...] = jnp.sum(p_ref[...], axis=0, keepdims=True)


def _tc_reduce(partial2d):
    tcb = 6400
    return pl.pallas_call(
        _tc_reduce_body,
        grid=(E // tcb,),
        in_specs=[pl.BlockSpec((FS, tcb), lambda i: (0, i))],
        out_specs=pl.BlockSpec((1, tcb), lambda i: (0, i)),
        out_shape=jax.ShapeDtypeStruct((1, E), jnp.float32),
    )(partial2d)


@jax.jit
def kernel(h, edge_index, e):
    del e
    hb = h.astype(jnp.bfloat16)
    packed = lax.bitcast_convert_type(hb.reshape(N, D // 2, 2), jnp.int32)
    h_grouped = packed.reshape(N, FS, W).transpose(1, 0, 2).reshape(FS, N * W)
    src = edge_index[0]
    dst = edge_index[1]
    partial = _sc_partials(h_grouped, src, dst)
    score = _tc_reduce(partial.reshape(FS, E))
    return score.reshape(E, 1)


# speedup vs baseline: 4.5367x; 1.2003x over previous
import jax
import jax.numpy as jnp
from jax import lax
from jax.experimental import pallas as pl
from jax.experimental.pallas import tpu as pltpu
from jax.experimental.pallas import tpu_sc as plsc

N = 10000
E = 320000
D = 128

FS = 8
W = 8
ES = 4
EPW = E // ES
CHUNK = 8000
NCH = EPW // CHUNK
L = 16


def _sc_body(hg_hbm, src_hbm, dst_hbm, part_hbm, hs, src_v, dst_v, out_v):
    c = lax.axis_index("c")
    s = lax.axis_index("s")
    fg = s % FS
    eg = c * 2 + s // FS
    pltpu.sync_copy(hg_hbm.at[fg], hs)

    def chunk_body(k, _):
        ch = eg * NCH + k
        off = pl.multiple_of(ch * CHUNK, CHUNK)
        pltpu.sync_copy(src_hbm.at[pl.ds(off, CHUNK)], src_v)
        pltpu.sync_copy(dst_hbm.at[pl.ds(off, CHUNK)], dst_v)

        @plsc.parallel_loop(0, CHUNK // L, unroll=4)
        def grp(i):
            b = pl.multiple_of(i * L, L)
            si = src_v[pl.ds(b, L)]
            di = dst_v[pl.ds(b, L)]
            acc = jnp.zeros((L,), jnp.float32)
            for j in range(W):
                wa = plsc.load_gather(hs, [si + j])
                wb = plsc.load_gather(hs, [di + j])
                a2 = plsc.bitcast(wa, jnp.bfloat16)
                b2 = plsc.bitcast(wb, jnp.bfloat16)
                p = a2 * b2
                pa, pb = plsc.unpack(
                    p,
                    format=plsc.PackFormat.INTERLEAVED,
                    preferred_element_type=jnp.float32,
                )
                acc = acc + pa + pb
            out_v[pl.ds(b, L)] = acc

        pltpu.sync_copy(out_v, part_hbm.at[fg, ch])
        return 0

    lax.fori_loop(0, NCH, chunk_body, 0)


def _sc_partials(h_grouped, src, dst):
    mesh = plsc.VectorSubcoreMesh(core_axis_name="c", subcore_axis_name="s")
    k = pl.kernel(
        _sc_body,
        out_type=jax.ShapeDtypeStruct((FS, ES * NCH, CHUNK), jnp.float32),
        mesh=mesh,
        compiler_params=pltpu.CompilerParams(needs_layout_passes=False),
        scratch_types=[
            pltpu.VMEM((N * W,), jnp.int32),
            pltpu.VMEM((CHUNK,), jnp.int32),
            pltpu.VMEM((CHUNK,), jnp.int32),
            pltpu.VMEM((CHUNK,), jnp.float32),
        ],
    )
    return k(h_grouped, src, dst)


def _tc_reduce_body(p_ref, o_ref):
    o_ref[...] = jnp.sum(p_ref[...], axis=0, keepdims=True)


def _tc_reduce(partial2d):
    tcb = 6400
    return pl.pallas_call(
        _tc_reduce_body,
        grid=(E // tcb,),
        in_specs=[pl.BlockSpec((FS, tcb), lambda i: (0, i))],
        out_specs=pl.BlockSpec((1, tcb), lambda i: (0, i)),
        out_shape=jax.ShapeDtypeStruct((1, E), jnp.float32),
    )(partial2d)


@jax.jit
def kernel(h, edge_index, e):
    del e
    hb = h.astype(jnp.bfloat16)
    packed = lax.bitcast_convert_type(hb.reshape(N, D // 2, 2), jnp.int32)
    h_grouped = packed.reshape(N, FS, W).transpose(1, 0, 2).reshape(FS, N * W)
    src = edge_index[0] * W
    dst = edge_index[1] * W
    partial = _sc_partials(h_grouped, src, dst)
    score = _tc_reduce(partial.reshape(FS, E))
    return score.reshape(E, 1)


# speedup vs baseline: 8.6344x; 1.9032x over previous
import jax
import jax.numpy as jnp
from jax import lax
from jax.experimental import pallas as pl
from jax.experimental.pallas import tpu as pltpu
from jax.experimental.pallas import tpu_sc as plsc

N = 10000
E = 320000
D = 128

FS = 8
W = 8
ES = 4
EPW = E // ES
CHUNK = 8000
NCH = EPW // CHUNK
L = 16


def _sc_body(hg_hbm, ei_hbm, part_hbm, hs, src_v, dst_v, out_v):
    c = lax.axis_index("c")
    s = lax.axis_index("s")
    fg = s % FS
    eg = c * 2 + s // FS
    pltpu.sync_copy(hg_hbm.at[fg], hs)

    def chunk_body(k, _):
        ch = eg * NCH + k
        off = pl.multiple_of(ch * CHUNK, CHUNK)
        pltpu.sync_copy(ei_hbm.at[pl.ds(off, CHUNK)], src_v)
        pltpu.sync_copy(ei_hbm.at[pl.ds(E + off, CHUNK)], dst_v)

        @plsc.parallel_loop(0, CHUNK // L, unroll=4)
        def grp(i):
            b = pl.multiple_of(i * L, L)
            si = src_v[pl.ds(b, L)]
            di = dst_v[pl.ds(b, L)]
            acc = jnp.zeros((L,), jnp.float32)
            for j in range(W):
                wa = plsc.load_gather(hs, [si + j * N])
                wb = plsc.load_gather(hs, [di + j * N])
                a2 = plsc.bitcast(wa, jnp.bfloat16)
                b2 = plsc.bitcast(wb, jnp.bfloat16)
                p = a2 * b2
                pa, pb = plsc.unpack(
                    p,
                    format=plsc.PackFormat.INTERLEAVED,
                    preferred_element_type=jnp.float32,
                )
                acc = acc + pa + pb
            out_v[pl.ds(b, L)] = acc

        pltpu.sync_copy(out_v, part_hbm.at[fg, ch])
        return 0

    lax.fori_loop(0, NCH, chunk_body, 0)


def _sc_partials(h_grouped, edge_index):
    mesh = plsc.VectorSubcoreMesh(core_axis_name="c", subcore_axis_name="s")
    k = pl.kernel(
        _sc_body,
        out_type=jax.ShapeDtypeStruct((FS, ES * NCH, CHUNK), jnp.float32),
        mesh=mesh,
        compiler_params=pltpu.CompilerParams(needs_layout_passes=False),
        scratch_types=[
            pltpu.VMEM((N * W,), jnp.int32),
            pltpu.VMEM((CHUNK,), jnp.int32),
            pltpu.VMEM((CHUNK,), jnp.int32),
            pltpu.VMEM((CHUNK,), jnp.float32),
        ],
    )
    return k(h_grouped, edge_index)


def _tc_reduce_body(p_ref, o_ref):
    o_ref[...] = jnp.sum(p_ref[...], axis=0, keepdims=True)


def _tc_reduce(partial2d):
    tcb = 6400
    return pl.pallas_call(
        _tc_reduce_body,
        grid=(E // tcb,),
        in_specs=[pl.BlockSpec((FS, tcb), lambda i: (0, i))],
        out_specs=pl.BlockSpec((1, tcb), lambda i: (0, i)),
        out_shape=jax.ShapeDtypeStruct((1, E), jnp.float32),
    )(partial2d)


@jax.jit
def kernel(h, edge_index, e):
    del e
    hb = h.astype(jnp.bfloat16)
    packed = lax.bitcast_convert_type(hb.reshape(N, D // 2, 2), jnp.int32)
    h_grouped = packed.T.reshape(FS, W * N)
    partial = _sc_partials(h_grouped, edge_index.reshape(2 * E))
    score = _tc_reduce(partial.reshape(FS, E))
    return score.reshape(E, 1)
